# Initial kernel scaffold; baseline (speedup 1.0000x reference)
import jax
import jax.numpy as jnp
from jax.experimental import pallas as pl


def kernel(batch, x, edge_index, edge_attr, edge_weight, params):
    raise NotImplementedError("write your pallas kernel here")


# speedup vs baseline: 1.2444x; 1.2444x over previous
import functools

import jax
import jax.numpy as jnp
from jax import lax
from jax.experimental import pallas as pl
from jax.experimental.pallas import tpu as pltpu
from jax.experimental.pallas import tpu_sc as plsc

N = 10000
E = 320000
NF = 128
EF = 16
H = 300
HP = 320
HH = 160
NG = 256
ODIM = 300

NS = 16
BE = 80
ET = E // NS
NBLK = ET // BE
NP = 10240
RPT = NP // NS

f32 = jnp.float32
bf16 = jnp.bfloat16


def _pad2(a, r, c):
    return jnp.pad(a, ((0, r - a.shape[0]), (0, c - a.shape[1])))


def _split_hi_lo(w):
    hi = w.astype(bf16)
    lo = (w - hi.astype(f32)).astype(bf16)
    return hi, lo


def _d3(a, w_hi, w_lo):
    del w_lo
    return jnp.dot(a.astype(bf16), w_hi, preferred_element_type=f32)


def _node_enc_body(x_ref, w0h, w0l, w1h, w1l, w2h, w2l, b0, b12, out_ref):
    x = x_ref[...]
    h = jnp.maximum(_d3(x, w0h[...], w0l[...]) + b0[...], 0.0)
    g0 = jnp.maximum(_d3(h, w1h[0], w1l[0]) + b12[0, 0], 0.0)
    g1 = jnp.maximum(_d3(h, w1h[1], w1l[1]) + b12[0, 1], 0.0)
    for nc in range(2):
        t = (_d3(g0, w2h[0, nc], w2l[0, nc])
             + _d3(g1, w2h[1, nc], w2l[1, nc]) + b12[1, nc])
        out_ref[nc] = jnp.maximum(t, 0.0)


_BEE = 5000


def _edge_enc_body(ea_ref, w0, w1, w2, b0, b12, out_ref):
    a = ea_ref[...].astype(bf16)
    h = jnp.maximum(jnp.dot(a, w0[...], preferred_element_type=f32)
                    + b0[...], 0.0).astype(bf16)
    g0 = jnp.maximum(jnp.dot(h, w1[0], preferred_element_type=f32)
                     + b12[0, 0], 0.0).astype(bf16)
    g1 = jnp.maximum(jnp.dot(h, w1[1], preferred_element_type=f32)
                     + b12[0, 1], 0.0).astype(bf16)
    for nc in range(2):
        t = (jnp.dot(g0, w2[0, nc], preferred_element_type=f32)
             + jnp.dot(g1, w2[1, nc], preferred_element_type=f32)
             + b12[1, nc])
        out_ref[nc] = jnp.maximum(t, 0.0)


_RB = 2000


def _layer_mm_body(h_ref, a_ref, whi, wlo, bv, out_ref):
    x0 = h_ref[0] + a_ref[0]
    x1 = h_ref[1] + a_ref[1]
    for nc in range(2):
        out_ref[nc] = (_d3(x0, whi[0, nc], wlo[0, nc])
                       + _d3(x1, whi[1, nc], wlo[1, nc]) + bv[0, nc])


def _layer_bn_body(t_ref, bv, out_ref, *, do_relu):
    for nc in range(2):
        t = t_ref[nc]
        mu = jnp.mean(t, axis=0, keepdims=True)
        var = jnp.mean(jnp.square(t - mu), axis=0, keepdims=True)
        y = (t - mu) * lax.rsqrt(var + 1e-5) * bv[1, nc] + bv[2, nc]
        if do_relu:
            y = jnp.maximum(y, 0.0)
        out_ref[nc] = y


def _pool_body(h_ref, b_ref, o0h, o0l, ob0, o1h, o1l, ob1, z_ref):
    gid = lax.broadcasted_iota(jnp.int32, (NG, N), 0)
    oh = (gid == b_ref[...]).astype(bf16)
    zs = []
    for kc in range(2):
        h = h_ref[kc]
        hh = h.astype(bf16)
        hl = (h - hh.astype(f32)).astype(bf16)
        zs.append(jnp.dot(oh, hh, preferred_element_type=f32)
                  + jnp.dot(oh, hl, preferred_element_type=f32))
    t0 = jnp.maximum(_d3(zs[0], o0h[0, 0], o0l[0, 0])
                     + _d3(zs[1], o0h[1, 0], o0l[1, 0]) + ob0[0], 0.0)
    t1 = jnp.maximum(_d3(zs[0], o0h[0, 1], o0l[0, 1])
                     + _d3(zs[1], o0h[1, 1], o0l[1, 1]) + ob0[1], 0.0)
    z_ref[...] = (_d3(t0, o1h[0], o1l[0])
                  + _d3(t1, o1h[1], o1l[1]) + ob1[...])


def _sc_msg_body(h_hbm, e_hbm, src_hbm, dst_hbm, w_hbm, out_hbm,
                 src_v, dst_v, h_v, e_v, w_s, agg_sh, sem):
    c = lax.axis_index("c")
    s = lax.axis_index("s")

    def _z(i, _):
        r = i // (HH // 16)
        k = i - r * (HH // 16)
        h_v[r, pl.ds(k * 16, 16)] = jnp.zeros((16,), f32)
        return 0
    lax.fori_loop(0, BE * (HH // 16), _z, 0)
    row0 = s * RPT
    for j in range(RPT // BE):
        pltpu.sync_copy(h_v, agg_sh.at[pl.ds(row0 + j * BE, BE)])

    plsc.subcore_barrier()
    off = c * N

    def _blk(i, _):
        base = s * ET + i * BE
        pltpu.sync_copy(src_hbm.at[pl.ds(base, BE)], src_v)
        pltpu.sync_copy(dst_hbm.at[pl.ds(base, BE)], dst_v)
        pltpu.sync_copy(w_hbm.at[pl.ds(base, BE)], w_s)

        def _ofs(k, _):
            sl = pl.ds(k * 16, 16)
            src_v[sl] = src_v[sl] + off
            return 0
        lax.fori_loop(0, BE // 16, _ofs, 0)

        gcp = pltpu.async_copy(h_hbm.at[src_v], h_v, sem)
        pltpu.sync_copy(e_hbm.at[pl.ds(c * E + base, BE)], e_v)
        gcp.wait()

        def _edge(j, _):
            jg = j // 16
            jl = j - jg * 16
            wv = w_s[pl.ds(jg * 16, 16)]
            w = wv.at[jnp.full((16,), jl, jnp.int32)].get(
                mode="promise_in_bounds")
            for k in range(HH // 16):
                sl = pl.ds(k * 16, 16)
                e_v[j, sl] = jnp.maximum(h_v[j, sl] + e_v[j, sl], 0.0) * w
            return 0
        lax.fori_loop(0, BE, _edge, 0)
        pltpu.sync_copy(e_v, agg_sh.at[dst_v], add=True)
        return 0
    lax.fori_loop(0, NBLK, _blk, 0)

    plsc.subcore_barrier()
    pltpu.sync_copy(agg_sh.at[pl.ds(row0, RPT)],
                    out_hbm.at[pl.ds(c * NP + row0, RPT)])


def _make_sc_msg():
    mesh = plsc.VectorSubcoreMesh(core_axis_name="c", subcore_axis_name="s",
                                  num_cores=2, num_subcores=NS)
    return pl.kernel(
        _sc_msg_body,
        out_type=jax.ShapeDtypeStruct((2 * NP, HH), f32),
        mesh=mesh,
        scratch_types=[
            pltpu.VMEM((BE,), jnp.int32),
            pltpu.VMEM((BE,), jnp.int32),
            pltpu.VMEM((BE, HH), f32),
            pltpu.VMEM((BE, HH), f32),
            pltpu.VMEM((BE,), f32),
            pltpu.VMEM_SHARED((NP, HH), f32),
            pltpu.SemaphoreType.DMA,
        ],
        compiler_params=pltpu.CompilerParams(use_tc_tiling_on_sc=False),
    )


def _enc_weights(Ws, bs, in_dim):
    w0h, w0l = _split_hi_lo(Ws[0])
    w1p = _pad2(Ws[1], Ws[1].shape[0], HP)
    w1h, w1l = _split_hi_lo(jnp.stack([w1p[:, :HH], w1p[:, HH:]]))
    w2p = _pad2(Ws[2], HP, HP)
    w2blk = jnp.stack([
        jnp.stack([w2p[:HH, :HH], w2p[:HH, HH:]]),
        jnp.stack([w2p[HH:, :HH], w2p[HH:, HH:]]),
    ])
    w2h, w2l = _split_hi_lo(w2blk)
    b0 = bs[0].reshape(1, in_dim)
    b1p = jnp.pad(bs[1], (0, HP - bs[1].shape[0]))
    b2p = jnp.pad(bs[2], (0, HP - bs[2].shape[0]))
    b12 = jnp.stack([
        jnp.stack([b1p[:HH].reshape(1, HH), b1p[HH:].reshape(1, HH)]),
        jnp.stack([b2p[:HH].reshape(1, HH), b2p[HH:].reshape(1, HH)]),
    ])
    return (w0h, w0l, w1h, w1l, w2h, w2l, b0, b12)


def _blk4(w):
    wp = _pad2(w, HP, HP)
    return jnp.stack([
        jnp.stack([wp[:HH, :HH], wp[:HH, HH:]]),
        jnp.stack([wp[HH:, :HH], wp[HH:, HH:]]),
    ])


def _halves(v):
    vp = jnp.pad(v, (0, HP - v.shape[0]))
    return jnp.stack([vp[:HH].reshape(1, HH), vp[HH:].reshape(1, HH)])


def kernel(batch, x, edge_index, edge_attr, edge_weight, params):
    nw = _enc_weights(params['atom_W'], params['atom_b'], NF)
    nwspec = [pl.BlockSpec(a.shape, lambda r, nd=a.ndim: (0,) * nd)
              for a in nw]
    h_st = pl.pallas_call(
        _node_enc_body,
        grid=(N // _RB,),
        in_specs=[pl.BlockSpec((_RB, NF), lambda r: (r, 0))] + nwspec,
        out_specs=pl.BlockSpec((2, _RB, HH), lambda r: (0, r, 0)),
        out_shape=jax.ShapeDtypeStruct((2, N, HH), f32),
    )(x, *nw)

    ew = _enc_weights(params['bond_W'], params['bond_b'], EF)
    (ew0h, ew0l, ew1h, ew1l, ew2h, ew2l, eb0, eb12) = ew
    nblk = E // _BEE
    wspec = lambda a: pl.BlockSpec(a.shape, lambda i: (0,) * a.ndim)
    e_st = pl.pallas_call(
        _edge_enc_body,
        grid=(nblk,),
        in_specs=[
            pl.BlockSpec((_BEE, EF), lambda i: (i, 0)),
            wspec(ew0h), wspec(ew1h), wspec(ew2h),
            wspec(eb0), wspec(eb12),
        ],
        out_specs=pl.BlockSpec((2, _BEE, HH), lambda i: (0, i, 0)),
        out_shape=jax.ShapeDtypeStruct((2, E, HH), f32),
    )(edge_attr, ew0h, ew1h, ew2h, eb0, eb12)

    sc_msg = _make_sc_msg()
    src = edge_index[0].astype(jnp.int32)
    dst = edge_index[1].astype(jnp.int32)
    wvec = edge_weight.reshape(E)
    e_flat = e_st.reshape(2 * E, HH)

    hcur = h_st
    for i in range(3):
        agg = sc_msg(hcur.reshape(2 * N, HH), e_flat, src, dst,
                     wvec).reshape(2, NP, HH)[:, :N]
        whi, wlo = _split_hi_lo(_blk4(params['conv_W'][i]))
        bv = jnp.stack([_halves(params['conv_b'][i]),
                        _halves(params['bn_g'][i]),
                        _halves(params['bn_b'][i])], axis=0)
        wspec2 = lambda a: pl.BlockSpec(a.shape, lambda r: (0,) * a.ndim)
        t = pl.pallas_call(
            _layer_mm_body,
            grid=(N // _RB,),
            in_specs=[
                pl.BlockSpec((2, _RB, HH), lambda r: (0, r, 0)),
                pl.BlockSpec((2, _RB, HH), lambda r: (0, r, 0)),
                wspec2(whi), wspec2(wlo), wspec2(bv),
            ],
            out_specs=pl.BlockSpec((2, _RB, HH), lambda r: (0, r, 0)),
            out_shape=jax.ShapeDtypeStruct((2, N, HH), f32),
        )(hcur, agg, whi, wlo, bv)
        hcur = pl.pallas_call(
            functools.partial(_layer_bn_body, do_relu=(i != 2)),
            out_shape=jax.ShapeDtypeStruct((2, N, HH), f32),
        )(t, bv)

    o0h, o0l = _split_hi_lo(_blk4(params['out_W'][0]))
    o1p = _pad2(params['out_W'][1], HP, HP)
    o1h, o1l = _split_hi_lo(jnp.stack([o1p[:HH], o1p[HH:]]))
    ob0 = _halves(params['out_b'][0])
    ob1 = jnp.pad(params['out_b'][1], (0, HP - ODIM)).reshape(1, HP)
    zfull = pl.pallas_call(
        _pool_body,
        out_shape=jax.ShapeDtypeStruct((NG, HP), f32),
    )(hcur, batch.astype(jnp.int32).reshape(1, N), o0h, o0l, ob0,
      o1h, o1l, ob1)

    z = zfull[:, :ODIM]
    node_emb = jnp.concatenate([hcur[0], hcur[1]], axis=1)[:, :H]
    return z, node_emb


# speedup vs baseline: 2.3852x; 1.9168x over previous
import functools

import jax
import jax.numpy as jnp
from jax import lax
from jax.experimental import pallas as pl
from jax.experimental.pallas import tpu as pltpu
from jax.experimental.pallas import tpu_sc as plsc

N = 10000
E = 320000
NF = 128
EF = 16
H = 300
HP = 320
HH = 160
NG = 256
ODIM = 300

NS = 16
BE = 80
ET = E // NS
NBLK = ET // BE
NP = 10240
RPT = NP // NS

f32 = jnp.float32
bf16 = jnp.bfloat16


def _pad2(a, r, c):
    return jnp.pad(a, ((0, r - a.shape[0]), (0, c - a.shape[1])))


def _bdot(a, w):
    return jnp.dot(a.astype(bf16), w, preferred_element_type=f32)


def _node_enc_body(x_ref, w0, w1, w2, b0, b12, out_ref):
    x = x_ref[...]
    h = jnp.maximum(_bdot(x, w0[...]) + b0[...], 0.0)
    g0 = jnp.maximum(_bdot(h, w1[0]) + b12[0, 0], 0.0)
    g1 = jnp.maximum(_bdot(h, w1[1]) + b12[0, 1], 0.0)
    for nc in range(2):
        t = _bdot(g0, w2[0, nc]) + _bdot(g1, w2[1, nc]) + b12[1, nc]
        y = jnp.maximum(t, 0.0)
        out_ref[2 * nc] = y[:, :QW]
        out_ref[2 * nc + 1] = y[:, QW:]


_BEE = 5000


def _edge_enc_body(ea_ref, w0, w1, w2, b0, b12, out_ref):
    a = ea_ref[...].astype(bf16)
    h = jnp.maximum(jnp.dot(a, w0[...], preferred_element_type=f32)
                    + b0[...], 0.0).astype(bf16)
    g0 = jnp.maximum(jnp.dot(h, w1[0], preferred_element_type=f32)
                     + b12[0, 0], 0.0).astype(bf16)
    g1 = jnp.maximum(jnp.dot(h, w1[1], preferred_element_type=f32)
                     + b12[0, 1], 0.0).astype(bf16)
    for nc in range(2):
        t = (jnp.dot(g0, w2[0, nc], preferred_element_type=f32)
             + jnp.dot(g1, w2[1, nc], preferred_element_type=f32)
             + b12[1, nc])
        y = jnp.maximum(t, 0.0)
        out_ref[2 * nc] = y[:, :QW]
        out_ref[2 * nc + 1] = y[:, QW:]


_RB = 2000


def _layer_mm_body(h_ref, a_ref, w4, bv, out_ref):
    xq = [h_ref[q] + a_ref[q] for q in range(4)]
    for nc in range(2):
        t = bv[0, nc]
        for q in range(4):
            t = t + _bdot(xq[q], w4[q, nc])
        out_ref[2 * nc] = t[:, :QW]
        out_ref[2 * nc + 1] = t[:, QW:]


def _layer_bn_body(t_ref, bv, out_ref, *, do_relu):
    t = t_ref[0]
    mu = jnp.mean(t, axis=0, keepdims=True)
    var = jnp.mean(jnp.square(t - mu), axis=0, keepdims=True)
    y = (t - mu) * lax.rsqrt(var + 1e-5) * bv[1, 0] + bv[2, 0]
    if do_relu:
        y = jnp.maximum(y, 0.0)
    out_ref[0] = y


def _pool_body(h_ref, b_ref, o0, ob0, o1, ob1, z_ref):
    gid = lax.broadcasted_iota(jnp.int32, (NG, N), 0)
    oh = (gid == b_ref[...]).astype(bf16)
    zs = []
    for q in range(4):
        h = h_ref[q]
        hh = h.astype(bf16)
        hl = (h - hh.astype(f32)).astype(bf16)
        zs.append(jnp.dot(oh, hh, preferred_element_type=f32)
                  + jnp.dot(oh, hl, preferred_element_type=f32))
    ts = []
    for nc in range(2):
        t = ob0[nc]
        for q in range(4):
            t = t + _bdot(zs[q], o0[q, nc])
        ts.append(jnp.maximum(t, 0.0))
    z_ref[...] = _bdot(ts[0], o1[0]) + _bdot(ts[1], o1[1]) + ob1[...]


NQ = 4
QW = HP // NQ
GB = 2
GE = GB * BE
CB = 10
CE = CB * BE
NCH = NBLK // CB
NGR = CB // GB


def _sc_msg_body(h_hbm, e_hbm, src_hbm, dst_hbm, w_hbm, out_hbm,
                 src_c, dst_c, w_c, h_v, e_v, agg_sh, sem0, sem1):
    c = lax.axis_index("c")
    s = lax.axis_index("s")
    sems = (sem0, sem1)
    row0 = s * RPT

    for qp in range(2):
        qi = c * 2 + qp

        def _z(i, _):
            r = i // (QW // 16)
            k = i - r * (QW // 16)
            h_v[0, r, pl.ds(k * 16, 16)] = jnp.zeros((16,), f32)
            return 0
        lax.fori_loop(0, GE * (QW // 16), _z, 0)
        for j in range(RPT // GE):
            pltpu.sync_copy(h_v.at[0], agg_sh.at[pl.ds(row0 + j * GE, GE)])

        plsc.subcore_barrier()
        off = qi * N

        def _chunk(ch, _):
            cbase = s * ET + ch * CE
            pltpu.sync_copy(src_hbm.at[pl.ds(cbase, CE)], src_c)
            pltpu.sync_copy(dst_hbm.at[s * NCH + ch], dst_c)
            pltpu.sync_copy(w_hbm.at[pl.ds(cbase, CE)], w_c)

            def _ofs(k, _):
                sl = pl.ds(k * 16, 16)
                src_c[sl] = src_c[sl] + off
                return 0
            lax.fori_loop(0, CE // 16, _ofs, 0)

            def _fire(g):
                slot = g % 2
                cps = []
                for b in range(GB):
                    blk = g * GB + b
                    cps.append(pltpu.async_copy(
                        h_hbm.at[src_c.at[pl.ds(blk * BE, BE)]],
                        h_v.at[slot, pl.ds(b * BE, BE)], sems[slot]))
                cps.append(pltpu.async_copy(
                    e_hbm.at[pl.ds(qi * E + cbase + g * GE, GE)],
                    e_v.at[slot], sems[slot]))
                return cps

            pend = {0: _fire(0), 1: None}
            for g in range(NGR):
                slot = g % 2
                if g + 1 < NGR:
                    pend[1 - slot] = _fire(g + 1)
                for cp in pend[slot]:
                    cp.wait()

                def _edge(jj, _):
                    for u in range(2):
                        j = jj * 2 + u
                        w16 = w_c[g * GE + j]
                        for k in range(QW // 16):
                            sl = pl.ds(k * 16, 16)
                            e_v[slot, j, sl] = (
                                jnp.maximum(h_v[slot, j, sl]
                                            + e_v[slot, j, sl], 0.0) * w16)
                    return 0
                lax.fori_loop(0, GE // 2, _edge, 0)

                scps = [pltpu.async_copy(
                    e_v.at[slot, pl.ds(b * BE, BE)],
                    agg_sh.at[dst_c.at[g * GB + b]], sems[slot], add=True)
                    for b in range(GB)]
                for cp in scps:
                    cp.wait()
            return 0
        lax.fori_loop(0, NCH, _chunk, 0)

        plsc.subcore_barrier()
        pltpu.sync_copy(agg_sh.at[pl.ds(row0, RPT)],
                        out_hbm.at[pl.ds(qi * NP + row0, RPT)])
        plsc.subcore_barrier()


def _make_sc_msg():
    mesh = plsc.VectorSubcoreMesh(core_axis_name="c", subcore_axis_name="s",
                                  num_cores=2, num_subcores=NS)
    return pl.kernel(
        _sc_msg_body,
        out_type=jax.ShapeDtypeStruct((NQ * NP, QW), f32),
        mesh=mesh,
        scratch_types=[
            pltpu.VMEM((CE,), jnp.int32),
            pltpu.VMEM((CB, BE), jnp.int32),
            pltpu.VMEM((CE, 16), f32),
            pltpu.VMEM((2, GE, QW), f32),
            pltpu.VMEM((2, GE, QW), f32),
            pltpu.VMEM_SHARED((NP, QW), f32),
            pltpu.SemaphoreType.DMA,
            pltpu.SemaphoreType.DMA,
        ],
        compiler_params=pltpu.CompilerParams(use_tc_tiling_on_sc=False),
    )


def _enc_weights(Ws, bs, in_dim):
    w0 = Ws[0].astype(bf16)
    w1p = _pad2(Ws[1], Ws[1].shape[0], HP)
    w1 = jnp.stack([w1p[:, :HH], w1p[:, HH:]]).astype(bf16)
    w2p = _pad2(Ws[2], HP, HP)
    w2 = jnp.stack([
        jnp.stack([w2p[:HH, :HH], w2p[:HH, HH:]]),
        jnp.stack([w2p[HH:, :HH], w2p[HH:, HH:]]),
    ]).astype(bf16)
    b0 = bs[0].reshape(1, in_dim)
    b1p = jnp.pad(bs[1], (0, HP - bs[1].shape[0]))
    b2p = jnp.pad(bs[2], (0, HP - bs[2].shape[0]))
    b12 = jnp.stack([
        jnp.stack([b1p[:HH].reshape(1, HH), b1p[HH:].reshape(1, HH)]),
        jnp.stack([b2p[:HH].reshape(1, HH), b2p[HH:].reshape(1, HH)]),
    ])
    return (w0, w1, w2, b0, b12)


def _blk42(w):
    wp = _pad2(w, HP, HP)
    return jnp.stack([
        jnp.stack([wp[q * QW:(q + 1) * QW, :HH],
                   wp[q * QW:(q + 1) * QW, HH:]])
        for q in range(4)
    ]).astype(bf16)


def _halves2(v):
    vp = jnp.pad(v, (0, HP - v.shape[0]))
    return jnp.stack([vp[:HH].reshape(1, HH), vp[HH:].reshape(1, HH)])


def _quarters(v):
    vp = jnp.pad(v, (0, HP - v.shape[0]))
    return jnp.stack([vp[q * QW:(q + 1) * QW].reshape(1, QW)
                      for q in range(4)])


def kernel(batch, x, edge_index, edge_attr, edge_weight, params):
    wspec = lambda a: pl.BlockSpec(a.shape, lambda i, nd=a.ndim: (0,) * nd)

    nw = _enc_weights(params['atom_W'], params['atom_b'], NF)
    h_st = pl.pallas_call(
        _node_enc_body,
        grid=(N // _RB,),
        in_specs=[pl.BlockSpec((_RB, NF), lambda r: (r, 0))]
        + [wspec(a) for a in nw],
        out_specs=pl.BlockSpec((4, _RB, QW), lambda r: (0, r, 0)),
        out_shape=jax.ShapeDtypeStruct((4, N, QW), f32),
    )(x, *nw)

    ew = _enc_weights(params['bond_W'], params['bond_b'], EF)
    e_st = pl.pallas_call(
        _edge_enc_body,
        grid=(E // _BEE,),
        in_specs=[pl.BlockSpec((_BEE, EF), lambda i: (i, 0))]
        + [wspec(a) for a in ew],
        out_specs=pl.BlockSpec((4, _BEE, QW), lambda i: (0, i, 0)),
        out_shape=jax.ShapeDtypeStruct((4, E, QW), f32),
    )(edge_attr, *ew)

    sc_msg = _make_sc_msg()
    src = edge_index[0].astype(jnp.int32)
    dst = edge_index[1].astype(jnp.int32).reshape(NS * NCH, CB, BE)
    wvec = jnp.broadcast_to(
        edge_weight.reshape(E, 1).astype(f32), (E, 16))
    e_flat = e_st.reshape(NQ * E, QW)

    hcur = h_st
    for i in range(3):
        agg = sc_msg(hcur.reshape(NQ * N, QW), e_flat, src, dst,
                     wvec).reshape(NQ, NP, QW)[:, :N]
        w4 = _blk42(params['conv_W'][i])
        bvh = jnp.stack([_halves2(params['conv_b'][i]),
                         _halves2(params['bn_g'][i]),
                         _halves2(params['bn_b'][i])])
        bvq = jnp.stack([_quarters(params['conv_b'][i]),
                         _quarters(params['bn_g'][i]),
                         _quarters(params['bn_b'][i])])
        t = pl.pallas_call(
            _layer_mm_body,
            grid=(N // _RB,),
            in_specs=[
                pl.BlockSpec((4, _RB, QW), lambda r: (0, r, 0)),
                pl.BlockSpec((4, _RB, QW), lambda r: (0, r, 0)),
                wspec(w4), wspec(bvh),
            ],
            out_specs=pl.BlockSpec((4, _RB, QW), lambda r: (0, r, 0)),
            out_shape=jax.ShapeDtypeStruct((4, N, QW), f32),
        )(hcur, agg, w4, bvh)
        hcur = pl.pallas_call(
            functools.partial(_layer_bn_body, do_relu=(i != 2)),
            grid=(4,),
            in_specs=[
                pl.BlockSpec((1, N, QW), lambda q: (q, 0, 0)),
                pl.BlockSpec((3, 1, 1, QW), lambda q: (0, q, 0, 0)),
            ],
            out_specs=pl.BlockSpec((1, N, QW), lambda q: (q, 0, 0)),
            out_shape=jax.ShapeDtypeStruct((4, N, QW), f32),
        )(t, bvq)

    o0 = _blk42(params['out_W'][0])
    o1p = _pad2(params['out_W'][1], HP, HP)
    o1 = jnp.stack([o1p[:HH], o1p[HH:]]).astype(bf16)
    ob0 = _halves2(params['out_b'][0])
    ob1 = jnp.pad(params['out_b'][1], (0, HP - ODIM)).reshape(1, HP)
    zfull = pl.pallas_call(
        _pool_body,
        out_shape=jax.ShapeDtypeStruct((NG, HP), f32),
    )(hcur, batch.astype(jnp.int32).reshape(1, N), o0, ob0, o1, ob1)

    z = zfull[:, :ODIM]
    node_emb = jnp.concatenate([hcur[0], hcur[1], hcur[2], hcur[3]],
                               axis=1)[:, :H]
    return z, node_emb
